# baseline (speedup 1.0000x reference)
import jax
import jax.numpy as jnp
from jax import lax
from jax.experimental import pallas as pl
from jax.experimental.pallas import tpu as pltpu
from jax.experimental.pallas import tpu_sc as plsc

VOCAB = 1000000
EMB = 32
B = 4096
L = 50

BK = 2048
NPART = 4
NBLK = 123
Q = NBLK * BK
TPAD = NPART * Q

NC = 2
NS = 16
HALF = B * L
CHUNK = 512
NCHUNK = HALF // CHUNK // NS
BPL = B // CHUNK


LASTB = VOCAB // BK
TAILK = LASTB - 3 * NBLK
TAIL0 = LASTB * BK


def _relayout_body(x0, x1, x2, x3, xt, y_ref):
    k = pl.program_id(0)
    x3v = jnp.where(k >= TAILK, xt[...], x3[...])
    y_ref[...] = jnp.concatenate(
        [x0[...].T, x1[...].T, x2[...].T, x3v.T], axis=1)


def _relayout_table(table_t, tail_pad):
    def in_spec(a):
        if a < 3:
            return pl.BlockSpec((EMB, BK), lambda k, a=a: (0, NBLK * a + k))
        return pl.BlockSpec(
            (EMB, BK), lambda k: (0, jnp.minimum(3 * NBLK + k, LASTB - 1)))

    tail_spec = pl.BlockSpec(
        (EMB, BK), lambda k: (0, jnp.maximum(k - TAILK, 0)))
    return pl.pallas_call(
        _relayout_body,
        grid=(NBLK,),
        in_specs=[in_spec(a) for a in range(NPART)] + [tail_spec],
        out_specs=pl.BlockSpec((BK, 128), lambda k: (k, 0)),
        out_shape=jax.ShapeDtypeStruct((Q, 128), jnp.float32),
    )(table_t, table_t, table_t, table_t, tail_pad)


def _transpose_chunk(rows_v, trans_v):
    iota16 = lax.iota(jnp.int32, 16)

    def per_e(e, carry):
        col = jnp.full((16,), e, jnp.int32)
        for j in range(CHUNK // 16):
            v = plsc.load_gather(rows_v, [iota16 + (16 * j), col])
            trans_v[e, pl.ds(16 * j, 16)] = v
        return carry

    lax.fori_loop(0, EMB, per_e, 0)


def _gather_body(table_hbm, ipost_hbm, iresp_hbm, out_hbm,
                 idx_v0, idx_v1, rows_v0, rows_v1, trans_v,
                 si0, si1, sg0, sg1, so):
    c = lax.axis_index("c")
    s = lax.axis_index("s")
    ids = (idx_v0, idx_v1)
    rows = (rows_v0, rows_v1)
    si = (si0, si1)
    sg = (sg0, sg1)

    def run(idx_hbm):
        def idx_at(k):
            return idx_hbm.at[pl.ds((s * NCHUNK + k) * CHUNK, CHUNK)]

        def out_at(k):
            chunk_id = s * NCHUNK + k
            l = chunk_id // BPL
            b0 = (chunk_id % BPL) * CHUNK
            return out_hbm.at[pl.ds((c * L + l) * EMB, EMB), pl.ds(b0, CHUNK)]

        def wait_out(k):
            pltpu.make_async_copy(trans_v, out_at(k), so).wait()

        def chunk_step(k, b, first, last, prefetch2):
            b1 = 1 - b
            pltpu.make_async_copy(table_hbm.at[ids[b]], rows[b], sg[b]).wait()
            if not last:
                pltpu.make_async_copy(idx_at(k + 1), ids[b1], si[b1]).wait()
                pltpu.async_copy(table_hbm.at[ids[b1]], rows[b1], sg[b1])
                if prefetch2:
                    pltpu.async_copy(idx_at(k + 2), ids[b], si[b])
            if not first:
                wait_out(k - 1)
            _transpose_chunk(rows[b], trans_v)
            pltpu.async_copy(trans_v, out_at(k), so)

        pltpu.async_copy(idx_at(0), ids[0], si[0])
        pltpu.async_copy(idx_at(1), ids[1], si[1])
        pltpu.make_async_copy(idx_at(0), ids[0], si[0]).wait()
        pltpu.async_copy(table_hbm.at[ids[0]], rows[0], sg[0])

        chunk_step(0, 0, first=True, last=False, prefetch2=True)

        def pair_body(t, carry):
            chunk_step(2 * t + 1, 1, first=False, last=False, prefetch2=True)
            chunk_step(2 * t + 2, 0, first=False, last=False, prefetch2=True)
            return carry

        lax.fori_loop(0, (NCHUNK - 2) // 2, pair_body, 0)

        chunk_step(NCHUNK - 2, 1, first=False, last=False, prefetch2=False)
        chunk_step(NCHUNK - 1, 0, first=False, last=True, prefetch2=False)
        wait_out(NCHUNK - 1)

    @pl.when(c == 0)
    def _():
        run(ipost_hbm)

    @pl.when(c == 1)
    def _():
        run(iresp_hbm)


def _embed_gather(table_lin, ipost, iresp):
    mesh = plsc.VectorSubcoreMesh(core_axis_name="c", subcore_axis_name="s")
    return pl.kernel(
        _gather_body,
        out_type=jax.ShapeDtypeStruct((2 * L * EMB, B), jnp.float32),
        mesh=mesh,
        scratch_types=[
            pltpu.VMEM((CHUNK,), jnp.int32),
            pltpu.VMEM((CHUNK,), jnp.int32),
            pltpu.VMEM((CHUNK, EMB), jnp.float32),
            pltpu.VMEM((CHUNK, EMB), jnp.float32),
            pltpu.VMEM((EMB, CHUNK), jnp.float32),
            pltpu.SemaphoreType.DMA,
            pltpu.SemaphoreType.DMA,
            pltpu.SemaphoreType.DMA,
            pltpu.SemaphoreType.DMA,
            pltpu.SemaphoreType.DMA,
        ],
        compiler_params=pltpu.CompilerParams(
            use_tc_tiling_on_sc=False, needs_layout_passes=False),
    )(table_lin, ipost, iresp)


@jax.jit
def _pointer_embedding(table, post, resp):
    table_t = table.T
    tail_pad = jnp.pad(
        table_t[:, TAIL0:], ((0, 0), (0, (NBLK - TAILK) * BK - (VOCAB - TAIL0))))
    table_lin = _relayout_table(table_t, tail_pad).reshape(TPAD, EMB)

    def to_lin_idx(x):
        v = x.T.reshape(-1)
        a = v // Q
        return 4 * (v - a * Q) + a

    g = _embed_gather(table_lin, to_lin_idx(post), to_lin_idx(resp))
    return g.reshape(2, L, EMB, B).transpose(0, 3, 1, 2)


def kernel(table, post, resp):
    return _pointer_embedding(table, post, resp)


# speedup vs baseline: 1.6256x; 1.6256x over previous
import jax
import jax.numpy as jnp
from jax import lax
from jax.experimental import pallas as pl
from jax.experimental.pallas import tpu as pltpu
from jax.experimental.pallas import tpu_sc as plsc

VOCAB = 1000000
EMB = 32
B = 4096
L = 50

BK = 2048
NPART = 4
NBLK = 123
Q = NBLK * BK
TPAD = NPART * Q

NC = 2
NS = 16
HALF = B * L
CHUNK = 512
NCHUNK = HALF // CHUNK // NS
BPL = B // CHUNK


LASTB = VOCAB // BK
TAILK = LASTB - 3 * NBLK
TAIL0 = LASTB * BK


def _relayout_body(x0, x1, x2, x3, xt, y_ref):
    k = pl.program_id(0)
    x3v = jnp.where(k >= TAILK, xt[...], x3[...])
    y_ref[...] = jnp.concatenate(
        [x0[...].T, x1[...].T, x2[...].T, x3v.T], axis=1)


def _relayout_table(table_t, tail_pad):
    def in_spec(a):
        if a < 3:
            return pl.BlockSpec((EMB, BK), lambda k, a=a: (0, NBLK * a + k))
        return pl.BlockSpec(
            (EMB, BK), lambda k: (0, jnp.minimum(3 * NBLK + k, LASTB - 1)))

    tail_spec = pl.BlockSpec(
        (EMB, BK), lambda k: (0, jnp.maximum(k - TAILK, 0)))
    return pl.pallas_call(
        _relayout_body,
        grid=(NBLK,),
        in_specs=[in_spec(a) for a in range(NPART)] + [tail_spec],
        out_specs=pl.BlockSpec((BK, 128), lambda k: (k, 0)),
        out_shape=jax.ShapeDtypeStruct((Q, 128), jnp.float32),
    )(table_t, table_t, table_t, table_t, tail_pad)


def _transpose_chunk(rows_v, trans_v):
    iota16 = lax.iota(jnp.int32, 16)

    def per_group(gidx, carry):
        r0 = gidx * 8
        for u in range(8):
            r = r0 + u
            col = jnp.full((16,), r, jnp.int32)
            v0 = rows_v[r, pl.ds(0, 16)]
            v1 = rows_v[r, pl.ds(16, 16)]
            plsc.store_scatter(trans_v, [iota16, col], v0)
            plsc.store_scatter(trans_v, [iota16 + 16, col], v1)
        return carry

    lax.fori_loop(0, CHUNK // 8, per_group, 0)


def _gather_body(table_hbm, ipost_hbm, iresp_hbm, out_hbm,
                 idx_v0, idx_v1, rows_v0, rows_v1, trans_v,
                 si0, si1, sg0, sg1, so):
    c = lax.axis_index("c")
    s = lax.axis_index("s")
    ids = (idx_v0, idx_v1)
    rows = (rows_v0, rows_v1)
    si = (si0, si1)
    sg = (sg0, sg1)

    def run(idx_hbm):
        def idx_at(k):
            return idx_hbm.at[pl.ds((s * NCHUNK + k) * CHUNK, CHUNK)]

        def out_at(k):
            chunk_id = s * NCHUNK + k
            l = chunk_id // BPL
            b0 = (chunk_id % BPL) * CHUNK
            return out_hbm.at[c, l, :, pl.ds(b0, CHUNK)]

        def trans_src():
            return trans_v.at[:, pl.ds(0, CHUNK)]

        def wait_out(k):
            pltpu.make_async_copy(trans_src(), out_at(k), so).wait()

        def chunk_step(k, b, first, last, prefetch2):
            b1 = 1 - b
            pltpu.make_async_copy(table_hbm.at[ids[b]], rows[b], sg[b]).wait()
            if not last:
                pltpu.make_async_copy(idx_at(k + 1), ids[b1], si[b1]).wait()
                pltpu.async_copy(table_hbm.at[ids[b1]], rows[b1], sg[b1])
                if prefetch2:
                    pltpu.async_copy(idx_at(k + 2), ids[b], si[b])
            if not first:
                wait_out(k - 1)
            _transpose_chunk(rows[b], trans_v)
            pltpu.async_copy(trans_src(), out_at(k), so)

        pltpu.async_copy(idx_at(0), ids[0], si[0])
        pltpu.async_copy(idx_at(1), ids[1], si[1])
        pltpu.make_async_copy(idx_at(0), ids[0], si[0]).wait()
        pltpu.async_copy(table_hbm.at[ids[0]], rows[0], sg[0])

        chunk_step(0, 0, first=True, last=False, prefetch2=True)

        def pair_body(t, carry):
            chunk_step(2 * t + 1, 1, first=False, last=False, prefetch2=True)
            chunk_step(2 * t + 2, 0, first=False, last=False, prefetch2=True)
            return carry

        lax.fori_loop(0, (NCHUNK - 2) // 2, pair_body, 0)

        chunk_step(NCHUNK - 2, 1, first=False, last=False, prefetch2=False)
        chunk_step(NCHUNK - 1, 0, first=False, last=True, prefetch2=False)
        wait_out(NCHUNK - 1)

    @pl.when(c == 0)
    def _():
        run(ipost_hbm)

    @pl.when(c == 1)
    def _():
        run(iresp_hbm)


def _embed_gather(table_lin, ipost, iresp):
    mesh = plsc.VectorSubcoreMesh(core_axis_name="c", subcore_axis_name="s")
    return pl.kernel(
        _gather_body,
        out_type=jax.ShapeDtypeStruct((2, L, EMB, B), jnp.float32),
        mesh=mesh,
        scratch_types=[
            pltpu.VMEM((CHUNK,), jnp.int32),
            pltpu.VMEM((CHUNK,), jnp.int32),
            pltpu.VMEM((CHUNK, EMB), jnp.float32),
            pltpu.VMEM((CHUNK, EMB), jnp.float32),
            pltpu.VMEM((EMB, CHUNK + 1), jnp.float32),
            pltpu.SemaphoreType.DMA,
            pltpu.SemaphoreType.DMA,
            pltpu.SemaphoreType.DMA,
            pltpu.SemaphoreType.DMA,
            pltpu.SemaphoreType.DMA,
        ],
        compiler_params=pltpu.CompilerParams(
            use_tc_tiling_on_sc=False, needs_layout_passes=False),
    )(table_lin, ipost, iresp)


@jax.jit
def _pointer_embedding(table, post, resp):
    table_t = table.T
    tail_pad = jnp.pad(
        table_t[:, TAIL0:], ((0, 0), (0, (NBLK - TAILK) * BK - (VOCAB - TAIL0))))
    table_lin = _relayout_table(table_t, tail_pad).reshape(TPAD, EMB)

    def to_lin_idx(x):
        v = x.T.reshape(-1)
        a = v // Q
        return 4 * (v - a * Q) + a

    g = _embed_gather(table_lin, to_lin_idx(post), to_lin_idx(resp))
    return g.transpose(0, 3, 1, 2)


def kernel(table, post, resp):
    return _pointer_embedding(table, post, resp)


# speedup vs baseline: 1.6308x; 1.0032x over previous
import jax
import jax.numpy as jnp
from jax import lax
from jax.experimental import pallas as pl
from jax.experimental.pallas import tpu as pltpu
from jax.experimental.pallas import tpu_sc as plsc

VOCAB = 1000000
EMB = 32
B = 4096
L = 50

BK = 2048
NPART = 4
NBLK = 123
Q = NBLK * BK
TPAD = NPART * Q

NC = 2
NS = 16
HALF = B * L
CHUNK = 512
NCHUNK = HALF // CHUNK // NS
BPL = B // CHUNK


LASTB = VOCAB // BK
TAILK = LASTB - 3 * NBLK
TAIL0 = LASTB * BK


def _relayout_body(x0, x1, x2, x3, xt, y_ref):
    k = pl.program_id(0)
    x3v = jnp.where(k >= TAILK, xt[...], x3[...])
    y_ref[:, pl.ds(0, EMB)] = x0[...].T
    y_ref[:, pl.ds(EMB, EMB)] = x1[...].T
    y_ref[:, pl.ds(2 * EMB, EMB)] = x2[...].T
    y_ref[:, pl.ds(3 * EMB, EMB)] = x3v.T


def _relayout_table(table_t, tail_pad):
    def in_spec(a):
        if a < 3:
            return pl.BlockSpec((EMB, BK), lambda k, a=a: (0, NBLK * a + k))
        return pl.BlockSpec(
            (EMB, BK), lambda k: (0, jnp.minimum(3 * NBLK + k, LASTB - 1)))

    tail_spec = pl.BlockSpec(
        (EMB, BK), lambda k: (0, jnp.maximum(k - TAILK, 0)))
    return pl.pallas_call(
        _relayout_body,
        grid=(NBLK,),
        in_specs=[in_spec(a) for a in range(NPART)] + [tail_spec],
        out_specs=pl.BlockSpec((BK, 128), lambda k: (k, 0)),
        out_shape=jax.ShapeDtypeStruct((Q, 128), jnp.float32),
    )(table_t, table_t, table_t, table_t, tail_pad)


def _transpose_chunk(rows_v, trans_v):
    iota16 = lax.iota(jnp.int32, 16)

    def per_group(gidx, carry):
        r0 = gidx * 8
        for u in range(8):
            r = r0 + u
            col = jnp.full((16,), r, jnp.int32)
            v0 = rows_v[r, pl.ds(0, 16)]
            v1 = rows_v[r, pl.ds(16, 16)]
            plsc.store_scatter(trans_v, [iota16, col], v0)
            plsc.store_scatter(trans_v, [iota16 + 16, col], v1)
        return carry

    lax.fori_loop(0, CHUNK // 8, per_group, 0)


def _gather_body(table_hbm, ipost_hbm, iresp_hbm, out_hbm,
                 idx_v0, idx_v1, rows_v0, rows_v1, trans_v,
                 si0, si1, sg0, sg1, so):
    c = lax.axis_index("c")
    s = lax.axis_index("s")
    ids = (idx_v0, idx_v1)
    rows = (rows_v0, rows_v1)
    si = (si0, si1)
    sg = (sg0, sg1)

    def run(idx_hbm):
        def idx_at(k):
            return idx_hbm.at[pl.ds((s * NCHUNK + k) * CHUNK, CHUNK)]

        def out_at(k):
            chunk_id = s * NCHUNK + k
            l = chunk_id // BPL
            b0 = (chunk_id % BPL) * CHUNK
            return out_hbm.at[c, l, :, pl.ds(b0, CHUNK)]

        def trans_src():
            return trans_v.at[:, pl.ds(0, CHUNK)]

        def wait_out(k):
            pltpu.make_async_copy(trans_src(), out_at(k), so).wait()

        def chunk_step(k, b, first, last, prefetch2):
            b1 = 1 - b
            pltpu.make_async_copy(table_hbm.at[ids[b]], rows[b], sg[b]).wait()
            if not last:
                pltpu.make_async_copy(idx_at(k + 1), ids[b1], si[b1]).wait()
                pltpu.async_copy(table_hbm.at[ids[b1]], rows[b1], sg[b1])
                if prefetch2:
                    pltpu.async_copy(idx_at(k + 2), ids[b], si[b])
            if not first:
                wait_out(k - 1)
            _transpose_chunk(rows[b], trans_v)
            pltpu.async_copy(trans_src(), out_at(k), so)

        pltpu.async_copy(idx_at(0), ids[0], si[0])
        pltpu.async_copy(idx_at(1), ids[1], si[1])
        pltpu.make_async_copy(idx_at(0), ids[0], si[0]).wait()
        pltpu.async_copy(table_hbm.at[ids[0]], rows[0], sg[0])

        chunk_step(0, 0, first=True, last=False, prefetch2=True)

        def pair_body(t, carry):
            chunk_step(2 * t + 1, 1, first=False, last=False, prefetch2=True)
            chunk_step(2 * t + 2, 0, first=False, last=False, prefetch2=True)
            return carry

        lax.fori_loop(0, (NCHUNK - 2) // 2, pair_body, 0)

        chunk_step(NCHUNK - 2, 1, first=False, last=False, prefetch2=False)
        chunk_step(NCHUNK - 1, 0, first=False, last=True, prefetch2=False)
        wait_out(NCHUNK - 1)

    @pl.when(c == 0)
    def _():
        run(ipost_hbm)

    @pl.when(c == 1)
    def _():
        run(iresp_hbm)


def _embed_gather(table_lin, ipost, iresp):
    mesh = plsc.VectorSubcoreMesh(core_axis_name="c", subcore_axis_name="s")
    return pl.kernel(
        _gather_body,
        out_type=jax.ShapeDtypeStruct((2, L, EMB, B), jnp.float32),
        mesh=mesh,
        scratch_types=[
            pltpu.VMEM((CHUNK,), jnp.int32),
            pltpu.VMEM((CHUNK,), jnp.int32),
            pltpu.VMEM((CHUNK, EMB), jnp.float32),
            pltpu.VMEM((CHUNK, EMB), jnp.float32),
            pltpu.VMEM((EMB, CHUNK + 1), jnp.float32),
            pltpu.SemaphoreType.DMA,
            pltpu.SemaphoreType.DMA,
            pltpu.SemaphoreType.DMA,
            pltpu.SemaphoreType.DMA,
            pltpu.SemaphoreType.DMA,
        ],
        compiler_params=pltpu.CompilerParams(
            use_tc_tiling_on_sc=False, needs_layout_passes=False),
    )(table_lin, ipost, iresp)


@jax.jit
def _pointer_embedding(table, post, resp):
    table_t = table.T
    tail_pad = jnp.pad(
        table_t[:, TAIL0:], ((0, 0), (0, (NBLK - TAILK) * BK - (VOCAB - TAIL0))))
    table_lin = _relayout_table(table_t, tail_pad).reshape(TPAD, EMB)

    def to_lin_idx(x):
        v = x.T.reshape(-1)
        a = v // Q
        return 4 * (v - a * Q) + a

    g = _embed_gather(table_lin, to_lin_idx(post), to_lin_idx(resp))
    return g.transpose(0, 3, 1, 2)


def kernel(table, post, resp):
    return _pointer_embedding(table, post, resp)
